# baseline (speedup 1.0000x reference)
import functools

import jax
import jax.numpy as jnp
import numpy as np
from jax import lax
from jax.experimental import pallas as pl
from jax.experimental.pallas import tpu as pltpu
from jax.experimental.pallas import tpu_sc as plsc

N = 100000
E = 3200000
D = 128
H = 64
C = 16
K = 10
ALPHA = 0.1

NP = 100096
WROWS = NP // 16
ZB = 184

NS = 16
GW = 256
SB = 4
TT = 392
EP = 2 * NS * TT * GW
NCH = TT // SB

_mesh = plsc.VectorSubcoreMesh(core_axis_name="c", subcore_axis_name="s")
_sc_params = pltpu.CompilerParams(use_tc_tiling_on_sc=False)
_Z = np.int32(0)


def _i32(v):
    return jnp.asarray(v, jnp.int32)


def _loop32(lo, hi):
    return pl.loop(jnp.int32(lo), jnp.int32(hi))


def _zero_acc(acc_sh, zbuf, sid, sem):
    @_loop32(0, ZB)
    def _(i):
        zbuf[pl.ds(i, 1), :] = jnp.zeros((1, C), jnp.float32)

    cps = [
        pltpu.async_copy(
            zbuf, acc_sh.at[pl.ds(sid * _i32(WROWS) + _i32(j * ZB), ZB)],
            sem) for j in range(WROWS // ZB)
    ]
    for cp in cps:
        cp.wait()


def _writeback(acc_sh, acc_hbm, cid, sid):
    pltpu.sync_copy(
        acc_sh.at[pl.ds(sid * _i32(WROWS), WROWS)],
        acc_hbm.at[pl.ds(cid * _i32(NP) + sid * _i32(WROWS), WROWS)])


@functools.partial(
    pl.kernel,
    out_type=jax.ShapeDtypeStruct((2 * NP, C), jnp.float32),
    mesh=_mesh,
    compiler_params=_sc_params,
    scratch_types=[
        pltpu.VMEM((SB, GW), jnp.int32),
        pltpu.VMEM((ZB, C), jnp.float32),
        pltpu.VMEM((GW, C), jnp.float32),
        pltpu.VMEM_SHARED((NP, C), jnp.float32),
        pltpu.SemaphoreType.DMA,
    ],
)
def _deg_sc(dst_hbm, acc_hbm, dstv, zbuf, ones, acc_sh, sem):
    cid = lax.axis_index("c")
    sid = lax.axis_index("s")
    _zero_acc(acc_sh, zbuf, sid, sem)

    @_loop32(0, GW)
    def _(i):
        ones[pl.ds(i, 1), :] = jnp.ones((1, C), jnp.float32)

    plsc.subcore_barrier()

    @_loop32(0, NCH)
    def _(ch):
        base = ((cid * _i32(NS) + sid) * _i32(NCH) + ch) * _i32(SB)
        pltpu.sync_copy(dst_hbm.at[pl.ds(base, SB)], dstv)

        cps = [
            pltpu.async_copy(ones, acc_sh.at[dstv.at[_i32(b)]], sem,
                             add=True) for b in range(SB)
        ]
        for cp in cps:
            cp.wait()

    plsc.subcore_barrier()
    _writeback(acc_sh, acc_hbm, cid, sid)


@functools.partial(
    pl.kernel,
    out_type=jax.ShapeDtypeStruct((2 * NP, C), jnp.float32),
    mesh=_mesh,
    compiler_params=_sc_params,
    scratch_types=[
        pltpu.VMEM((SB, GW), jnp.int32),
        pltpu.VMEM((SB, GW), jnp.int32),
        pltpu.VMEM((SB * GW, C), jnp.float32),
        pltpu.VMEM((ZB, C), jnp.float32),
        pltpu.VMEM_SHARED((NP, C), jnp.float32),
        pltpu.SemaphoreType.DMA,
        pltpu.SemaphoreType.DMA,
    ],
)
def _prop_sc(z_hbm, srcs_hbm, dst_hbm, acc_hbm, srcv, dstv, rows, zbuf,
             acc_sh, gsem, ssem):
    cid = lax.axis_index("c")
    sid = lax.axis_index("s")
    _zero_acc(acc_sh, zbuf, sid, gsem)
    plsc.subcore_barrier()

    @_loop32(0, NCH)
    def _(ch):
        base = ((cid * _i32(NS) + sid) * _i32(NCH) + ch) * _i32(SB)
        pltpu.sync_copy(srcs_hbm.at[pl.ds(base, SB)], srcv)
        pltpu.sync_copy(dst_hbm.at[pl.ds(base, SB)], dstv)

        ga = [
            pltpu.async_copy(z_hbm.at[srcv.at[_i32(b)]],
                             rows.at[pl.ds(_i32(b * GW), GW)], gsem)
            for b in range(SB // 2)
        ]
        for cp in ga:
            cp.wait()
        sa = [
            pltpu.async_copy(rows.at[pl.ds(_i32(b * GW), GW)],
                             acc_sh.at[dstv.at[_i32(b)]], ssem, add=True)
            for b in range(SB // 2)
        ]
        gb = [
            pltpu.async_copy(z_hbm.at[srcv.at[_i32(b)]],
                             rows.at[pl.ds(_i32(b * GW), GW)], gsem)
            for b in range(SB // 2, SB)
        ]
        for cp in gb:
            cp.wait()
        sb = [
            pltpu.async_copy(rows.at[pl.ds(_i32(b * GW), GW)],
                             acc_sh.at[dstv.at[_i32(b)]], ssem, add=True)
            for b in range(SB // 2, SB)
        ]
        for cp in sa + sb:
            cp.wait()

    plsc.subcore_barrier()
    _writeback(acc_sh, acc_hbm, cid, sid)


RM = 3128


def _mlp_body(x_ref, w1_ref, b1_ref, w2_ref, b2_ref, h_ref):
    i = pl.program_id(0)
    h1 = jnp.maximum(
        jnp.dot(x_ref[...], w1_ref[...],
                preferred_element_type=jnp.float32) + b1_ref[...], 0.0)
    h2 = jnp.maximum(
        jnp.dot(h1, w2_ref[...],
                preferred_element_type=jnp.float32) + b2_ref[...], 0.0)
    rows = i * RM + lax.broadcasted_iota(jnp.int32, (RM, 1), 0)
    h_ref[...] = jnp.where(rows < N, h2, 0.0)


def _mlp(x_p, W1, b1, W2, b2):
    return pl.pallas_call(
        _mlp_body,
        grid=(NP // RM,),
        in_specs=[
            pl.BlockSpec((RM, D), lambda i: (i, _Z)),
            pl.BlockSpec((D, H), lambda i: (_Z, _Z)),
            pl.BlockSpec((1, H), lambda i: (_Z, _Z)),
            pl.BlockSpec((H, C), lambda i: (_Z, _Z)),
            pl.BlockSpec((1, C), lambda i: (_Z, _Z)),
        ],
        out_specs=pl.BlockSpec((RM, C), lambda i: (i, _Z)),
        out_shape=jax.ShapeDtypeStruct((NP, C), jnp.float32),
    )(x_p, W1, b1.reshape(1, H), W2, b2.reshape(1, C))


RP = 6256
_NB = np.int32(NP // RP)


def _prep_body(dga_ref, dgb_ref, h_ref, dis_ref, d2_ref, z_ref):
    i = pl.program_id(0)
    deg = dga_ref[:, 0:1] + dgb_ref[:, 0:1] + 1.0
    rows = i * RP + lax.broadcasted_iota(jnp.int32, (RP, 1), 0)
    valid = rows < N
    dis = jnp.where(valid, lax.rsqrt(deg), 0.0)
    dis_ref[...] = dis
    d2_ref[...] = jnp.where(valid, 1.0 / deg, 0.0)
    z_ref[...] = dis * h_ref[...]


def _prep(deg_acc, h):
    return pl.pallas_call(
        _prep_body,
        grid=(NP // RP,),
        in_specs=[
            pl.BlockSpec((RP, C), lambda i: (i, _Z)),
            pl.BlockSpec((RP, C), lambda i: (i + _NB, _Z)),
            pl.BlockSpec((RP, C), lambda i: (i, _Z)),
        ],
        out_specs=[
            pl.BlockSpec((RP, 1), lambda i: (i, _Z)),
            pl.BlockSpec((RP, 1), lambda i: (i, _Z)),
            pl.BlockSpec((RP, C), lambda i: (i, _Z)),
        ],
        out_shape=[
            jax.ShapeDtypeStruct((NP, 1), jnp.float32),
            jax.ShapeDtypeStruct((NP, 1), jnp.float32),
            jax.ShapeDtypeStruct((NP, C), jnp.float32),
        ],
    )(deg_acc, deg_acc, h)


def _combine_body(aga_ref, agb_ref, o_ref, h_ref, dis_ref, d2_ref, on_ref,
                  zn_ref):
    agg = aga_ref[...] + agb_ref[...]
    on = (1.0 - ALPHA) * (dis_ref[...] * agg
                          + d2_ref[...] * o_ref[...]) + ALPHA * h_ref[...]
    on_ref[...] = on
    zn_ref[...] = dis_ref[...] * on


def _combine(agg, out, h, dis, d2):
    return pl.pallas_call(
        _combine_body,
        grid=(NP // RP,),
        in_specs=[
            pl.BlockSpec((RP, C), lambda i: (i, _Z)),
            pl.BlockSpec((RP, C), lambda i: (i + _NB, _Z)),
            pl.BlockSpec((RP, C), lambda i: (i, _Z)),
            pl.BlockSpec((RP, C), lambda i: (i, _Z)),
            pl.BlockSpec((RP, 1), lambda i: (i, _Z)),
            pl.BlockSpec((RP, 1), lambda i: (i, _Z)),
        ],
        out_specs=[
            pl.BlockSpec((RP, C), lambda i: (i, _Z)),
            pl.BlockSpec((RP, C), lambda i: (i, _Z)),
        ],
        out_shape=[
            jax.ShapeDtypeStruct((NP, C), jnp.float32),
            jax.ShapeDtypeStruct((NP, C), jnp.float32),
        ],
    )(agg, agg, out, h, dis, d2)


def kernel(x, edge_index, W1, b1, W2, b2):
    x = x.astype(jnp.float32)
    ei = edge_index.astype(jnp.int32)
    pad = jnp.full((EP - E,), N, jnp.int32)
    srcs = jnp.concatenate([ei[0], pad]).reshape(2 * NS * TT, GW)
    dst = jnp.concatenate([ei[1], pad]).reshape(2 * NS * TT, GW)
    x_p = jnp.pad(x, ((0, NP - N), (0, 0)))

    h = _mlp(x_p, W1.astype(jnp.float32), b1.astype(jnp.float32),
             W2.astype(jnp.float32), b2.astype(jnp.float32))
    deg_acc = _deg_sc(dst)
    dis, d2, z = _prep(deg_acc, h)

    out = h
    for _ in range(K):
        agg = _prop_sc(z, srcs, dst)
        out, z = _combine(agg, out, h, dis, d2)
    return out[:N].astype(jnp.float64)
